# baseline (speedup 1.0000x reference)
import functools

import jax
import jax.numpy as jnp
from jax import lax
from jax.experimental import pallas as pl
from jax.experimental.pallas import tpu as pltpu
from jax.experimental.pallas import tpu_sc as plsc

N = 320000
S = 10000
D = 128
NC = 2
NS = 16
NW = NC * NS
S_PAD = NW * 320
SEG = 320
ACC_ROWS = SEG + 8
CHUNK = 128
DIVC = 80


def _body(x_hbm, idx_hbm, starts_hbm, out_hbm,
          xb0, xb1, ib0, ib1, startsbuf, ssums, ends,
          sx0, sx1, si0, si1, ss0, ss1):
    c = lax.axis_index("c")
    s = lax.axis_index("s")
    wid = s * NC + c

    zero16 = jnp.zeros((16,), jnp.float32)
    iota16 = lax.iota(jnp.int32, 16)
    sbase = s * ACC_ROWS

    def frow(i, _):
        for j in range(8):
            xb0[i, pl.ds(16 * j, 16)] = zero16
        return 0
    lax.fori_loop(0, CHUNK, frow, 0)
    for k in range(2):
        pltpu.sync_copy(xb0, ssums.at[pl.ds(sbase + k * CHUNK, CHUNK)])
    pltpu.sync_copy(xb0.at[pl.ds(0, 64 + 8)],
                    ssums.at[pl.ds(sbase + 2 * CHUNK, 64 + 8)])

    def erow(i, _):
        ends[i] = 0
        return 0
    lax.fori_loop(0, SEG + 8, erow, 0)

    pltpu.sync_copy(starts_hbm, startsbuf)
    sv = startsbuf[pl.ds(wid, 16)]
    start = sv[0]
    end = sv[1]
    astart = (start // 8) * 8
    nwin = (end - astart + (CHUNK - 1)) // CHUNK
    npairs = (nwin + 1) // 2

    def woff(ci):
        return pl.multiple_of(
            jnp.minimum(astart + ci * CHUNK, N - CHUNK), 8)

    def dma_start(ci, xb, ib, sx, si):
        off = woff(ci)
        pltpu.async_copy(x_hbm.at[pl.ds(off, CHUNK)], xb, sx)
        pltpu.async_copy(idx_hbm.at[pl.ds(off, CHUNK)], ib, si)

    def dma_wait(ci, xb, ib, sx, si):
        off = woff(ci)
        pltpu.make_async_copy(x_hbm.at[pl.ds(off, CHUNK)], xb, sx).wait()
        pltpu.make_async_copy(idx_hbm.at[pl.ds(off, CHUNK)], ib, si).wait()

    def transform(ci, ib):
        off = woff(ci)
        lo = jnp.maximum(start, astart + ci * CHUNK)
        hi = jnp.minimum(end, astart + ci * CHUNK + CHUNK)
        for j in range(CHUNK // 16):
            iv = ib[pl.ds(16 * j, 16)]
            rows = off + 16 * j + iota16
            valid = (rows >= lo) & (rows < hi)
            tlv = jnp.where(valid, iv - SEG * wid, SEG)
            ib[pl.ds(16 * j, 16)] = tlv + sbase
            for k in range(16):
                ends[tlv[k]] = off + (16 * j + k + 1)

    def scat_start(xb, ib, ss):
        pltpu.async_copy(xb, ssums.at[ib], ss, add=True)

    def scat_wait(xb, ib, ss):
        pltpu.make_async_copy(xb, ssums.at[ib], ss).wait()

    dma_start(0, xb0, ib0, sx0, si0)
    dma_start(1, xb1, ib1, sx1, si1)

    def pair(p, _):
        ci0 = 2 * p
        dma_wait(ci0, xb0, ib0, sx0, si0)
        transform(ci0, ib0)
        scat_start(xb0, ib0, ss0)
        dma_wait(ci0 + 1, xb1, ib1, sx1, si1)
        transform(ci0 + 1, ib1)
        scat_start(xb1, ib1, ss1)
        scat_wait(xb0, ib0, ss0)
        dma_start(ci0 + 2, xb0, ib0, sx0, si0)
        scat_wait(xb1, ib1, ss1)
        dma_start(ci0 + 3, xb1, ib1, sx1, si1)
        return 0
    lax.fori_loop(0, npairs, pair, 0)
    dma_wait(2 * npairs, xb0, ib0, sx0, si0)
    dma_wait(2 * npairs + 1, xb1, ib1, sx1, si1)

    pm0 = start

    def divide_chunk(k, pm_in):
        pltpu.sync_copy(ssums.at[pl.ds(sbase + k * DIVC, DIVC)],
                        xb1.at[pl.ds(0, DIVC)])

        def drow(i, pm):
            e = ends[k * DIVC + i]
            pm_new = jnp.maximum(pm, e)
            cntf = (pm_new - pm).astype(jnp.float32)
            inv16 = 1.0 / jnp.maximum(jnp.broadcast_to(cntf, (16,)), 1.0)
            for j in range(8):
                sl = pl.ds(16 * j, 16)
                xb1[i, sl] = xb1[i, sl] * inv16
            return pm_new
        pm_out = lax.fori_loop(0, DIVC, drow, pm_in)

        @pl.when(wid * SEG + k * DIVC < S)
        def _():
            pltpu.sync_copy(xb1.at[pl.ds(0, DIVC)],
                            out_hbm.at[pl.ds(wid * SEG + k * DIVC, DIVC)])
        return pm_out

    lax.fori_loop(0, 4, divide_chunk, pm0)


_segmean = pl.kernel(
    _body,
    out_type=jax.ShapeDtypeStruct((S, D), jnp.float32),
    mesh=plsc.VectorSubcoreMesh(core_axis_name="c", subcore_axis_name="s"),
    scratch_types=[
        pltpu.VMEM((CHUNK, D), jnp.float32),
        pltpu.VMEM((CHUNK, D), jnp.float32),
        pltpu.VMEM((CHUNK,), jnp.int32),
        pltpu.VMEM((CHUNK,), jnp.int32),
        pltpu.VMEM((48,), jnp.int32),
        pltpu.VMEM_SHARED((NS * ACC_ROWS, D), jnp.float32),
        pltpu.SMEM((SEG + 8,), jnp.int32),
        pltpu.SemaphoreType.DMA,
        pltpu.SemaphoreType.DMA,
        pltpu.SemaphoreType.DMA,
        pltpu.SemaphoreType.DMA,
        pltpu.SemaphoreType.DMA,
        pltpu.SemaphoreType.DMA,
    ],
)


def kernel(x, index):
    bounds = jnp.arange(0, S_PAD + 1, SEG, dtype=jnp.int32)
    starts = jnp.sum(index[None, :] < bounds[:, None], axis=1, dtype=jnp.int32)
    starts = jnp.pad(starts, (0, 48 - starts.shape[0]))
    return _segmean(x, index, starts)


# speedup vs baseline: 1.2757x; 1.2757x over previous
import functools

import jax
import jax.numpy as jnp
from jax import lax
from jax.experimental import pallas as pl
from jax.experimental.pallas import tpu as pltpu
from jax.experimental.pallas import tpu_sc as plsc

N = 320000
S = 10000
D = 128
NC = 2
NS = 16
NW = NC * NS
S_PAD = NW * 320
SEG = 320
ACC_ROWS = SEG + 8
CHUNK = 128
DIVC = 80


def _body(x_hbm, idx_hbm, starts_hbm, out_hbm,
          xb0, xb1, ib0, ib1, startsbuf, ssums, ends,
          sx0, sx1, si0, si1, ss0, ss1):
    c = lax.axis_index("c")
    s = lax.axis_index("s")
    wid = s * NC + c

    zero16 = jnp.zeros((16,), jnp.float32)
    iota16 = lax.iota(jnp.int32, 16)
    sbase = s * ACC_ROWS

    def frow(i, _):
        for j in range(8):
            xb0[i, pl.ds(16 * j, 16)] = zero16
        return 0
    lax.fori_loop(0, CHUNK, frow, 0)
    for k in range(2):
        pltpu.sync_copy(xb0, ssums.at[pl.ds(sbase + k * CHUNK, CHUNK)])
    pltpu.sync_copy(xb0.at[pl.ds(0, 64 + 8)],
                    ssums.at[pl.ds(sbase + 2 * CHUNK, 64 + 8)])

    def erow(i, _):
        ends[i] = 0
        return 0
    lax.fori_loop(0, SEG + 8, erow, 0)

    pltpu.sync_copy(starts_hbm, startsbuf)
    sv = startsbuf[pl.ds(wid, 16)]
    start = sv[0]
    end = sv[1]
    astart = (start // 8) * 8
    nwin = (end - astart + (CHUNK - 1)) // CHUNK
    npairs = (nwin + 1) // 2

    def woff(ci):
        return pl.multiple_of(
            jnp.minimum(astart + ci * CHUNK, N - CHUNK), 8)

    def dma_start(ci, xb, ib, sx, si):
        off = woff(ci)
        pltpu.async_copy(x_hbm.at[pl.ds(off, CHUNK)], xb, sx)
        pltpu.async_copy(idx_hbm.at[pl.ds(off, CHUNK)], ib, si)

    def dma_wait(ci, xb, ib, sx, si):
        off = woff(ci)
        pltpu.make_async_copy(x_hbm.at[pl.ds(off, CHUNK)], xb, sx).wait()
        pltpu.make_async_copy(idx_hbm.at[pl.ds(off, CHUNK)], ib, si).wait()

    def transform(ci, ib):
        off = woff(ci)
        lo = jnp.maximum(start, astart + ci * CHUNK)
        hi = jnp.minimum(end, astart + ci * CHUNK + CHUNK)
        for j in range(CHUNK // 16):
            iv = ib[pl.ds(16 * j, 16)]
            rows = off + 16 * j + iota16
            valid = (rows >= lo) & (rows < hi)
            tlv = jnp.where(valid, iv - SEG * wid, SEG)
            ib[pl.ds(16 * j, 16)] = tlv + sbase
            for k in range(16):
                ends[tlv[k]] = off + (16 * j + k + 1)

    dma_start(0, xb0, ib0, sx0, si0)

    def pair(p, _):
        ci0 = 2 * p
        dma_start(ci0 + 1, xb1, ib1, sx1, si1)
        dma_wait(ci0, xb0, ib0, sx0, si0)
        transform(ci0, ib0)
        pltpu.sync_copy(xb0, ssums.at[ib0], add=True)
        dma_start(ci0 + 2, xb0, ib0, sx0, si0)
        dma_wait(ci0 + 1, xb1, ib1, sx1, si1)
        transform(ci0 + 1, ib1)
        pltpu.sync_copy(xb1, ssums.at[ib1], add=True)
        return 0
    lax.fori_loop(0, npairs, pair, 0)
    dma_wait(2 * npairs, xb0, ib0, sx0, si0)

    pm0 = start

    def divide_chunk(k, pm_in):
        pltpu.sync_copy(ssums.at[pl.ds(sbase + k * DIVC, DIVC)],
                        xb1.at[pl.ds(0, DIVC)])

        def drow(i, pm):
            e = ends[k * DIVC + i]
            pm_new = jnp.maximum(pm, e)
            cntf = (pm_new - pm).astype(jnp.float32)
            inv16 = 1.0 / jnp.maximum(jnp.broadcast_to(cntf, (16,)), 1.0)
            for j in range(8):
                sl = pl.ds(16 * j, 16)
                xb1[i, sl] = xb1[i, sl] * inv16
            return pm_new
        pm_out = lax.fori_loop(0, DIVC, drow, pm_in)

        @pl.when(wid * SEG + k * DIVC < S)
        def _():
            pltpu.sync_copy(xb1.at[pl.ds(0, DIVC)],
                            out_hbm.at[pl.ds(wid * SEG + k * DIVC, DIVC)])
        return pm_out

    lax.fori_loop(0, 4, divide_chunk, pm0)


_segmean = pl.kernel(
    _body,
    out_type=jax.ShapeDtypeStruct((S, D), jnp.float32),
    mesh=plsc.VectorSubcoreMesh(core_axis_name="c", subcore_axis_name="s"),
    scratch_types=[
        pltpu.VMEM((CHUNK, D), jnp.float32),
        pltpu.VMEM((CHUNK, D), jnp.float32),
        pltpu.VMEM((CHUNK,), jnp.int32),
        pltpu.VMEM((CHUNK,), jnp.int32),
        pltpu.VMEM((48,), jnp.int32),
        pltpu.VMEM_SHARED((NS * ACC_ROWS, D), jnp.float32),
        pltpu.SMEM((SEG + 8,), jnp.int32),
        pltpu.SemaphoreType.DMA,
        pltpu.SemaphoreType.DMA,
        pltpu.SemaphoreType.DMA,
        pltpu.SemaphoreType.DMA,
        pltpu.SemaphoreType.DMA,
        pltpu.SemaphoreType.DMA,
    ],
)


def kernel(x, index):
    bounds = jnp.arange(0, S_PAD + 1, SEG, dtype=jnp.int32)
    starts = jnp.sum(index[None, :] < bounds[:, None], axis=1, dtype=jnp.int32)
    starts = jnp.pad(starts, (0, 48 - starts.shape[0]))
    return _segmean(x, index, starts)
